# Initial kernel scaffold; baseline (speedup 1.0000x reference)
import jax
import jax.numpy as jnp
from jax.experimental import pallas as pl


def kernel(center_emb, node_embs, neighbor_idx, neighbor_conf, Wq, Wk, Wg, bg, gamma, beta):
    raise NotImplementedError("write your pallas kernel here")


# speedup vs baseline: 2.8147x; 2.8147x over previous
import functools

import jax
import jax.numpy as jnp
from jax import lax
from jax.experimental import pallas as pl
from jax.experimental.pallas import tpu as pltpu
from jax.experimental.pallas import tpu_sc as plsc

_B, _K, _N, _D, _A = 16384, 16, 50000, 256, 64
_NW = 32
_CH = 128
_NC = (_B * _K) // (_NW * _CH)


def _sc_gather(table, idx3):
    mesh = plsc.VectorSubcoreMesh(core_axis_name="c", subcore_axis_name="s")

    @functools.partial(
        pl.kernel,
        out_type=jax.ShapeDtypeStruct((_B * _K, _D), jnp.float32),
        mesh=mesh,
        scratch_types=[
            pltpu.VMEM((_NC, _CH), jnp.int32),
            pltpu.VMEM((_CH, _D), jnp.float32),
            pltpu.VMEM((_CH, _D), jnp.float32),
            pltpu.SemaphoreType.DMA,
            pltpu.SemaphoreType.DMA,
        ],
    )
    def k(table_hbm, idx_hbm, out_hbm, idx_v, rows0, rows1, sem0, sem1):
        wid = lax.axis_index("s") * 2 + lax.axis_index("c")
        base = wid * _NC * _CH
        pltpu.sync_copy(idx_hbm.at[wid], idx_v)
        bufs = (rows0, rows1)
        sems = (sem0, sem1)
        pltpu.async_copy(table_hbm.at[idx_v.at[0]], rows0, sem0)

        @pl.loop(0, _NC)
        def _(c):
            slot = lax.rem(c, 2)

            @pl.when(c + 1 < _NC)
            def _():
                nxt = lax.rem(c + 1, 2)
                for j in range(2):
                    @pl.when(nxt == j)
                    def _():
                        pltpu.async_copy(
                            table_hbm.at[idx_v.at[c + 1]], bufs[j], sems[j])

            for j in range(2):
                @pl.when(slot == j)
                def _():
                    pltpu.make_async_copy(
                        table_hbm.at[idx_v.at[c]], bufs[j], sems[j]).wait()
                    pltpu.sync_copy(
                        bufs[j], out_hbm.at[pl.ds(base + c * _CH, _CH)])

    return k(table, idx3)


def _tc_attention(rows3, center, conf, Wq, Wk, Wg1, Wg2, bg2, gamma2, beta2):
    RB = 256
    grid = (_B // RB,)

    def body(rows_ref, center_ref, conf_ref, wq_ref, wk_ref, wg1_ref,
             wg2_ref, bg_ref, g_ref, b_ref, out_ref):
        rows2 = rows_ref[...].reshape(RB * _K, _D)
        center = center_ref[...]
        q = jnp.dot(center, wq_ref[...], preferred_element_type=jnp.float32)
        k2 = jnp.dot(rows2, wk_ref[...], preferred_element_type=jnp.float32)
        k3 = k2.reshape(RB, _K, _A)
        scores = jnp.sum(k3 * q[:, None, :], axis=-1) * (1.0 / _K ** 0.5)
        scores = scores + jnp.maximum(jnp.log(conf_ref[...]), -10.0)
        m = jnp.max(scores, axis=-1, keepdims=True)
        e = jnp.exp(scores - m)
        w = e / jnp.sum(e, axis=-1, keepdims=True)
        ctx = jnp.sum(w[:, :, None] * rows_ref[...], axis=1)
        gs = (jnp.dot(center, wg1_ref[...], preferred_element_type=jnp.float32)
              + jnp.dot(ctx, wg2_ref[...], preferred_element_type=jnp.float32)
              + bg_ref[0, 0])
        gate = 1.0 / (1.0 + jnp.exp(-gs))
        outv = gate * center + (1.0 - gate) * ctx
        mean = jnp.mean(outv, axis=-1, keepdims=True)
        cent = outv - mean
        var = jnp.mean(cent * cent, axis=-1, keepdims=True)
        normed = cent * lax.rsqrt(var + 1e-5)
        out_ref[...] = normed * g_ref[...] + b_ref[...]

    return pl.pallas_call(
        body,
        grid=grid,
        in_specs=[
            pl.BlockSpec((RB, _K, _D), lambda i: (i, 0, 0)),
            pl.BlockSpec((RB, _D), lambda i: (i, 0)),
            pl.BlockSpec((RB, _K), lambda i: (i, 0)),
            pl.BlockSpec((_D, _A), lambda i: (0, 0)),
            pl.BlockSpec((_D, _A), lambda i: (0, 0)),
            pl.BlockSpec((_D, 1), lambda i: (0, 0)),
            pl.BlockSpec((_D, 1), lambda i: (0, 0)),
            pl.BlockSpec((1, 1), lambda i: (0, 0)),
            pl.BlockSpec((1, _D), lambda i: (0, 0)),
            pl.BlockSpec((1, _D), lambda i: (0, 0)),
        ],
        out_specs=pl.BlockSpec((RB, _D), lambda i: (i, 0)),
        out_shape=jax.ShapeDtypeStruct((_B, _D), jnp.float32),
        compiler_params=pltpu.CompilerParams(
            dimension_semantics=("arbitrary",),
        ),
    )(rows3, center, conf, Wq, Wk, Wg1, Wg2, bg2, gamma2, beta2)


def kernel(center_emb, node_embs, neighbor_idx, neighbor_conf, Wq, Wk, Wg,
           bg, gamma, beta):
    idx3 = neighbor_idx.reshape(_NW, _NC, _CH)
    rows = _sc_gather(node_embs, idx3)
    rows3 = rows.reshape(_B, _K, _D)
    Wg1 = Wg[:_D]
    Wg2 = Wg[_D:]
    bg2 = bg.reshape(1, 1)
    gamma2 = gamma.reshape(1, _D)
    beta2 = beta.reshape(1, _D)
    return _tc_attention(rows3, center_emb, neighbor_conf, Wq, Wk, Wg1, Wg2,
                         bg2, gamma2, beta2)


# speedup vs baseline: 4.0204x; 1.4283x over previous
import functools

import jax
import jax.numpy as jnp
from jax import lax
from jax.experimental import pallas as pl
from jax.experimental.pallas import tpu as pltpu
from jax.experimental.pallas import tpu_sc as plsc

_B, _K, _N, _D, _A = 16384, 16, 50000, 256, 64
_NW = 32
_CH = 128
_NC = (_B * _K) // (_NW * _CH)


def _sc_gather(table, idx3):
    mesh = plsc.VectorSubcoreMesh(core_axis_name="c", subcore_axis_name="s")

    @functools.partial(
        pl.kernel,
        out_type=jax.ShapeDtypeStruct((_B * _K, _D), jnp.float32),
        mesh=mesh,
        scratch_types=[
            pltpu.VMEM((_NC, _CH), jnp.int32),
            pltpu.VMEM((_CH, _D), jnp.float32),
            pltpu.VMEM((_CH, _D), jnp.float32),
            pltpu.SemaphoreType.DMA,
            pltpu.SemaphoreType.DMA,
        ],
    )
    def k(table_hbm, idx_hbm, out_hbm, idx_v, rows0, rows1, sem0, sem1):
        wid = lax.axis_index("s") * 2 + lax.axis_index("c")
        base = wid * _NC * _CH
        pltpu.sync_copy(idx_hbm.at[wid], idx_v)
        bufs = (rows0, rows1)
        sems = (sem0, sem1)
        pltpu.async_copy(table_hbm.at[idx_v.at[0]], rows0, sem0)

        @pl.loop(0, _NC)
        def _(c):
            slot = lax.rem(c, 2)

            @pl.when(c + 1 < _NC)
            def _():
                nxt = lax.rem(c + 1, 2)
                for j in range(2):
                    @pl.when(nxt == j)
                    def _():
                        pltpu.async_copy(
                            table_hbm.at[idx_v.at[c + 1]], bufs[j], sems[j])

            for j in range(2):
                @pl.when(slot == j)
                def _():
                    pltpu.make_async_copy(
                        table_hbm.at[idx_v.at[c]], bufs[j], sems[j]).wait()
                    pltpu.sync_copy(
                        bufs[j], out_hbm.at[pl.ds(base + c * _CH, _CH)])

    return k(table, idx3)


def _tc_attention(rows3, center, conf, Wq, Wk, Wg1, Wg2, bg2, gamma2, beta2):
    RB = 256
    grid = (_B // RB,)

    def body(rows_ref, center_ref, conf_ref, wq_ref, wk_ref, wg1_ref,
             wg2_ref, bg_ref, g_ref, b_ref, out_ref):
        rows2 = rows_ref[...].reshape(RB * _K, _D)
        center = center_ref[...]
        q = jnp.dot(center, wq_ref[...], preferred_element_type=jnp.float32)
        k2 = jnp.dot(rows2, wk_ref[...], preferred_element_type=jnp.float32)
        k3 = k2.reshape(RB, _K, _A)
        scores = jnp.sum(k3 * q[:, None, :], axis=-1) * (1.0 / _K ** 0.5)
        st = scores.T
        m = jnp.max(st, axis=0, keepdims=True)
        e = jnp.maximum(conf_ref[...], 4.5399929762484854e-05) * jnp.exp(st - m)
        wt = e / jnp.sum(e, axis=0, keepdims=True)
        w = wt.T
        ctx = jnp.sum(w[:, :, None] * rows_ref[...], axis=1)
        gs = (jnp.dot(center, wg1_ref[...], preferred_element_type=jnp.float32)
              + jnp.dot(ctx, wg2_ref[...], preferred_element_type=jnp.float32)
              + bg_ref[0, 0])
        gate = 1.0 / (1.0 + jnp.exp(-gs))
        outv = gate * center + (1.0 - gate) * ctx
        mean = jnp.mean(outv, axis=-1, keepdims=True)
        cent = outv - mean
        var = jnp.mean(cent * cent, axis=-1, keepdims=True)
        normed = cent * lax.rsqrt(var + 1e-5)
        out_ref[...] = normed * g_ref[...] + b_ref[...]

    return pl.pallas_call(
        body,
        grid=grid,
        in_specs=[
            pl.BlockSpec((RB, _K, _D), lambda i: (i, 0, 0)),
            pl.BlockSpec((RB, _D), lambda i: (i, 0)),
            pl.BlockSpec((_K, RB), lambda i: (0, i)),
            pl.BlockSpec((_D, _A), lambda i: (0, 0)),
            pl.BlockSpec((_D, _A), lambda i: (0, 0)),
            pl.BlockSpec((_D, 1), lambda i: (0, 0)),
            pl.BlockSpec((_D, 1), lambda i: (0, 0)),
            pl.BlockSpec((1, 1), lambda i: (0, 0)),
            pl.BlockSpec((1, _D), lambda i: (0, 0)),
            pl.BlockSpec((1, _D), lambda i: (0, 0)),
        ],
        out_specs=pl.BlockSpec((RB, _D), lambda i: (i, 0)),
        out_shape=jax.ShapeDtypeStruct((_B, _D), jnp.float32),
        compiler_params=pltpu.CompilerParams(
            dimension_semantics=("arbitrary",),
        ),
    )(rows3, center, conf, Wq, Wk, Wg1, Wg2, bg2, gamma2, beta2)


def kernel(center_emb, node_embs, neighbor_idx, neighbor_conf, Wq, Wk, Wg,
           bg, gamma, beta):
    idx3 = neighbor_idx.reshape(_NW, _NC, _CH)
    rows = _sc_gather(node_embs, idx3)
    rows3 = rows.reshape(_B, _K, _D)
    Wg1 = Wg[:_D]
    Wg2 = Wg[_D:]
    bg2 = bg.reshape(1, 1)
    gamma2 = gamma.reshape(1, _D)
    beta2 = beta.reshape(1, _D)
    conf_t = neighbor_conf.T
    return _tc_attention(rows3, center_emb, conf_t, Wq, Wk, Wg1, Wg2,
                         bg2, gamma2, beta2)


# speedup vs baseline: 4.2202x; 1.0497x over previous
import functools

import jax
import jax.numpy as jnp
from jax import lax
from jax.experimental import pallas as pl
from jax.experimental.pallas import tpu as pltpu
from jax.experimental.pallas import tpu_sc as plsc

_B, _K, _N, _D, _A = 16384, 16, 50000, 256, 64
_NW = 32
_CH = 128


def _sc_gather(table, idx3, nc):
    mesh = plsc.VectorSubcoreMesh(core_axis_name="c", subcore_axis_name="s")

    @functools.partial(
        pl.kernel,
        out_type=jax.ShapeDtypeStruct((_NW * nc * _CH, _D), jnp.float32),
        mesh=mesh,
        scratch_types=[
            pltpu.VMEM((nc, _CH), jnp.int32),
            pltpu.VMEM((_CH, _D), jnp.float32),
            pltpu.VMEM((_CH, _D), jnp.float32),
            pltpu.SemaphoreType.DMA,
            pltpu.SemaphoreType.DMA,
        ],
    )
    def k(table_hbm, idx_hbm, out_hbm, idx_v, rows0, rows1, sem0, sem1):
        wid = lax.axis_index("s") * 2 + lax.axis_index("c")
        base = wid * nc * _CH
        pltpu.sync_copy(idx_hbm.at[wid], idx_v)
        bufs = (rows0, rows1)
        sems = (sem0, sem1)
        pltpu.async_copy(table_hbm.at[idx_v.at[0]], rows0, sem0)

        @pl.loop(0, nc)
        def _(c):
            slot = lax.rem(c, 2)

            @pl.when(c + 1 < nc)
            def _():
                nxt = lax.rem(c + 1, 2)
                for j in range(2):
                    @pl.when(nxt == j)
                    def _():
                        pltpu.async_copy(
                            table_hbm.at[idx_v.at[c + 1]], bufs[j], sems[j])

            for j in range(2):
                @pl.when(slot == j)
                def _():
                    pltpu.make_async_copy(
                        table_hbm.at[idx_v.at[c]], bufs[j], sems[j]).wait()
                    pltpu.sync_copy(
                        bufs[j], out_hbm.at[pl.ds(base + c * _CH, _CH)])

    return k(table, idx3)


def _tc_attention(rows3, center, conf, Wq, Wk, Wg1, Wg2, bg2, gamma2, beta2):
    RB = 256
    nb = rows3.shape[0]
    grid = (nb // RB,)

    def body(rows_ref, center_ref, conf_ref, wq_ref, wk_ref, wg1_ref,
             wg2_ref, bg_ref, g_ref, b_ref, out_ref):
        rows2 = rows_ref[...].reshape(RB * _K, _D)
        center = center_ref[...]
        q = jnp.dot(center, wq_ref[...], preferred_element_type=jnp.float32)
        k2 = jnp.dot(rows2, wk_ref[...], preferred_element_type=jnp.float32)
        k3 = k2.reshape(RB, _K, _A)
        scores = jnp.sum(k3 * q[:, None, :], axis=-1) * (1.0 / _K ** 0.5)
        st = scores.T
        m = jnp.max(st, axis=0, keepdims=True)
        e = jnp.maximum(conf_ref[...], 4.5399929762484854e-05) * jnp.exp(st - m)
        wt = e / jnp.sum(e, axis=0, keepdims=True)
        w = wt.T
        ctx = jnp.sum(w[:, :, None] * rows_ref[...], axis=1)
        gs = (jnp.dot(center, wg1_ref[...], preferred_element_type=jnp.float32)
              + jnp.dot(ctx, wg2_ref[...], preferred_element_type=jnp.float32)
              + bg_ref[0, 0])
        gate = 1.0 / (1.0 + jnp.exp(-gs))
        outv = gate * center + (1.0 - gate) * ctx
        mean = jnp.mean(outv, axis=-1, keepdims=True)
        cent = outv - mean
        var = jnp.mean(cent * cent, axis=-1, keepdims=True)
        normed = cent * lax.rsqrt(var + 1e-5)
        out_ref[...] = normed * g_ref[...] + b_ref[...]

    return pl.pallas_call(
        body,
        grid=grid,
        in_specs=[
            pl.BlockSpec((RB, _K, _D), lambda i: (i, 0, 0)),
            pl.BlockSpec((RB, _D), lambda i: (i, 0)),
            pl.BlockSpec((_K, RB), lambda i: (0, i)),
            pl.BlockSpec((_D, _A), lambda i: (0, 0)),
            pl.BlockSpec((_D, _A), lambda i: (0, 0)),
            pl.BlockSpec((_D, 1), lambda i: (0, 0)),
            pl.BlockSpec((_D, 1), lambda i: (0, 0)),
            pl.BlockSpec((1, 1), lambda i: (0, 0)),
            pl.BlockSpec((1, _D), lambda i: (0, 0)),
            pl.BlockSpec((1, _D), lambda i: (0, 0)),
        ],
        out_specs=pl.BlockSpec((RB, _D), lambda i: (i, 0)),
        out_shape=jax.ShapeDtypeStruct((nb, _D), jnp.float32),
        compiler_params=pltpu.CompilerParams(
            dimension_semantics=("arbitrary",),
        ),
    )(rows3, center, conf, Wq, Wk, Wg1, Wg2, bg2, gamma2, beta2)


def kernel(center_emb, node_embs, neighbor_idx, neighbor_conf, Wq, Wk, Wg,
           bg, gamma, beta):
    G = 4
    BG = _B // G
    nc = (BG * _K) // (_NW * _CH)
    Wg1 = Wg[:_D]
    Wg2 = Wg[_D:]
    bg2 = bg.reshape(1, 1)
    gamma2 = gamma.reshape(1, _D)
    beta2 = beta.reshape(1, _D)
    conf_t = neighbor_conf.T
    rows_g = []
    for g in range(G):
        idx3 = neighbor_idx[g * BG:(g + 1) * BG].reshape(_NW, nc, _CH)
        rows_g.append(_sc_gather(node_embs, idx3, nc))
    outs = []
    for g in range(G):
        rows3 = rows_g[g].reshape(BG, _K, _D)
        outs.append(_tc_attention(
            rows3, center_emb[g * BG:(g + 1) * BG],
            conf_t[:, g * BG:(g + 1) * BG], Wq, Wk, Wg1, Wg2,
            bg2, gamma2, beta2))
    return jnp.concatenate(outs, axis=0)
